# baseline (speedup 1.0000x reference)
import functools

import jax
import jax.numpy as jnp
from jax import lax
from jax.experimental import pallas as pl
from jax.experimental.pallas import tpu as pltpu
from jax.experimental.pallas import tpu_sc as plsc

N = 100000
D = 128
NWIDTH = 16
H = 128
G = 512

TILE = 1024
NB = -(-N // TILE)
NPADDED = NB * TILE
W = 64

CH = 128
NPART = NB * W
NCHUNK = NPART // CH
NC = 2
NS = 16
NWORK = NC * NS
KMAX = -(-NCHUNK // NWORK)
ACCROWS = 1024
ZROWS = ACCROWS // NS
GROWS = G // NS


def _pre_body(x_ref, v_ref, b_ref, W1_ref, b1_ref, W2_ref, b2_ref,
              part_ref, idx_ref, resid_ref, racc_ref):
    i = pl.program_id(0)

    @pl.when(i == 0)
    def _init():
        racc_ref[...] = jnp.zeros_like(racc_ref)

    xv = x_ref[...]
    v = v_ref[...]
    inv = jnp.sqrt(v[:, 0, :] ** 2 + v[:, 1, :] ** 2 + v[:, 2, :] ** 2)
    h = (jax.lax.dot(xv.astype(jnp.bfloat16), W1_ref[0:D, :].astype(jnp.bfloat16),
                     preferred_element_type=jnp.float32)
         + jax.lax.dot(inv.astype(jnp.bfloat16),
                       W1_ref[D:D + NWIDTH, :].astype(jnp.bfloat16),
                       preferred_element_type=jnp.float32)
         + b1_ref[...])
    h = h * jax.nn.sigmoid(h)
    h = jax.lax.dot(h.astype(jnp.bfloat16), W2_ref[...].astype(jnp.bfloat16),
                    preferred_element_type=jnp.float32) + b2_ref[...]
    rvalid = lax.broadcasted_iota(jnp.int32, (TILE, 1), 0) + i * TILE
    h = jnp.where(rvalid < N, h, 0.0)
    hb = h.astype(jnp.bfloat16)

    ids = b_ref[0]
    s0 = ids[0, 0]
    local = ids - s0
    jwin = lax.broadcasted_iota(jnp.int32, (W, TILE), 0)
    ohw = (jwin == local).astype(jnp.bfloat16)
    part_ref[0] = jax.lax.dot(ohw, hb, preferred_element_type=jnp.float32)
    idx_ref[0] = s0 + lax.broadcasted_iota(jnp.int32, (1, W), 1)

    @pl.when(jnp.max(ids) - s0 >= W)
    def _residual():
        rows = lax.broadcasted_iota(jnp.int32, (G, TILE), 0)
        ohr = ((rows == ids) & (local >= W)).astype(jnp.bfloat16)
        racc_ref[...] += jax.lax.dot(ohr, hb, preferred_element_type=jnp.float32)

    @pl.when(i == NB - 1)
    def _flush():
        resid_ref[...] = racc_ref[...]


def _segsum_body(part_hbm, idx_hbm, out_hbm, rows_v, idx_v, stage_v, acc_sh):
    cid = lax.axis_index("c")
    sid = lax.axis_index("s")
    wid = sid * NC + cid

    for r in range(ZROWS):
        for j in range(H // 16):
            stage_v[r, pl.ds(j * 16, 16)] = jnp.zeros((16,), jnp.float32)
    pltpu.sync_copy(stage_v, acc_sh.at[pl.ds(sid * ZROWS, ZROWS)])
    plsc.subcore_barrier()

    for k in range(KMAX):
        c = wid + NWORK * k

        @pl.when(c < NCHUNK)
        def _chunk():
            base = c * CH
            pltpu.sync_copy(idx_hbm.at[pl.ds(base, CH)], idx_v.at[k])
            pltpu.sync_copy(part_hbm.at[pl.ds(base, CH), :], rows_v)
            pltpu.sync_copy(rows_v, acc_sh.at[idx_v.at[k]], add=True)

    plsc.subcore_barrier()
    pltpu.sync_copy(acc_sh.at[pl.ds(sid * GROWS, GROWS)],
                    out_hbm.at[cid, pl.ds(sid * GROWS, GROWS)])


def _post_body(pp_ref, resid_ref, W3_ref, b3_ref, W4_ref, b4_ref, out_ref):
    p = pp_ref[0] + pp_ref[1] + resid_ref[...]
    g = jax.lax.dot(p, W3_ref[...], preferred_element_type=jnp.float32) + b3_ref[...]
    g = g * jax.nn.sigmoid(g)
    out_ref[...] = (jax.lax.dot(g, W4_ref[...], preferred_element_type=jnp.float32)
                    + b4_ref[...])


@jax.jit
def kernel(x, V, batch, W1, b1, W2, b2, W3, b3, W4, b4):
    batch_p = jnp.concatenate(
        [batch, jnp.full((NPADDED - N,), G, jnp.int32)]).reshape(NB, 1, TILE)

    full = lambda *s: pl.BlockSpec(s, lambda i: (0,) * len(s))
    parts, idxs, resid = pl.pallas_call(
        _pre_body,
        grid=(NB,),
        in_specs=[
            pl.BlockSpec((TILE, D), lambda i: (i, 0)),
            pl.BlockSpec((TILE, 3, NWIDTH), lambda i: (i, 0, 0)),
            pl.BlockSpec((1, 1, TILE), lambda i: (i, 0, 0)),
            full(D + NWIDTH, H),
            full(1, H),
            full(H, H),
            full(1, H),
        ],
        out_specs=[
            pl.BlockSpec((1, W, H), lambda i: (i, 0, 0)),
            pl.BlockSpec((1, 1, W), lambda i: (i, 0, 0)),
            pl.BlockSpec((G, H), lambda i: (0, 0)),
        ],
        out_shape=[
            jax.ShapeDtypeStruct((NB, W, H), jnp.float32),
            jax.ShapeDtypeStruct((NB, 1, W), jnp.int32),
            jax.ShapeDtypeStruct((G, H), jnp.float32),
        ],
        scratch_shapes=[pltpu.VMEM((G, H), jnp.float32)],
        compiler_params=pltpu.CompilerParams(
            dimension_semantics=("arbitrary",),
        ),
    )(x, V, batch_p, W1, b1.reshape(1, H), W2, b2.reshape(1, H))

    mesh = plsc.VectorSubcoreMesh(core_axis_name="c", subcore_axis_name="s")
    segsum = functools.partial(
        pl.kernel,
        mesh=mesh,
        out_type=jax.ShapeDtypeStruct((NC, G, H), jnp.float32),
        scratch_types=[
            pltpu.VMEM((CH, H), jnp.float32),
            pltpu.VMEM((KMAX, CH), jnp.int32),
            pltpu.VMEM((ZROWS, H), jnp.float32),
            pltpu.VMEM_SHARED((ACCROWS, H), jnp.float32),
        ],
    )(_segsum_body)
    pooled2 = segsum(parts.reshape(NPART, H), idxs.reshape(NPART))

    out = pl.pallas_call(
        _post_body,
        in_specs=[
            pl.BlockSpec((NC, G, H), lambda: (0, 0, 0)),
            pl.BlockSpec((G, H), lambda: (0, 0)),
            pl.BlockSpec((H, H), lambda: (0, 0)),
            pl.BlockSpec((1, H), lambda: (0, 0)),
            pl.BlockSpec((H, 1), lambda: (0, 0)),
            pl.BlockSpec((1, 1), lambda: (0, 0)),
        ],
        out_specs=pl.BlockSpec((G, 1), lambda: (0, 0)),
        out_shape=jax.ShapeDtypeStruct((G, 1), jnp.float32),
    )(pooled2, resid, W3, b3.reshape(1, H), W4, b4.reshape(1, 1))
    return out


# speedup vs baseline: 2.6682x; 2.6682x over previous
import functools

import jax
import jax.numpy as jnp
from jax import lax
from jax.experimental import pallas as pl
from jax.experimental.pallas import tpu as pltpu
from jax.experimental.pallas import tpu_sc as plsc

N = 100000
D = 128
NWIDTH = 16
H = 128
G = 512

TILE = 1024
NB = -(-N // TILE)
NPADDED = NB * TILE
W = 64

CH = 128
NPART = NB * W
NCHUNK = NPART // CH
NC = 2
NS = 16
NWORK = NC * NS
KMAX = -(-NCHUNK // NWORK)
ACCROWS = 1024
ZROWS = ACCROWS // NS
GROWS = G // NS


def _pre_body(x_ref, v_ref, b_ref, W1_ref, b1_ref, W2_ref, b2_ref,
              part_ref, idx_ref, resid_ref, racc_ref):
    i = pl.program_id(0)

    @pl.when(i == 0)
    def _init():
        racc_ref[...] = jnp.zeros_like(racc_ref)

    xv = x_ref[...]
    v = v_ref[...].astype(jnp.float32)
    inv = jnp.sqrt(v[:, 0:16] ** 2 + v[:, 16:32] ** 2 + v[:, 32:48] ** 2)
    h = (jax.lax.dot(xv.astype(jnp.bfloat16), W1_ref[0:D, :].astype(jnp.bfloat16),
                     preferred_element_type=jnp.float32)
         + jax.lax.dot(inv.astype(jnp.bfloat16),
                       W1_ref[D:D + NWIDTH, :].astype(jnp.bfloat16),
                       preferred_element_type=jnp.float32)
         + b1_ref[...])
    h = h * jax.nn.sigmoid(h)
    h = jax.lax.dot(h.astype(jnp.bfloat16), W2_ref[...].astype(jnp.bfloat16),
                    preferred_element_type=jnp.float32) + b2_ref[...]
    rvalid = lax.broadcasted_iota(jnp.int32, (TILE, 1), 0) + i * TILE
    h = jnp.where(rvalid < N, h, 0.0)
    hb = h.astype(jnp.bfloat16)

    ids = b_ref[0]
    s0 = ids[0, 0]
    local = ids - s0
    jwin = lax.broadcasted_iota(jnp.int32, (W, TILE), 0)
    ohw = (jwin == local).astype(jnp.bfloat16)
    part_ref[0] = jax.lax.dot(ohw, hb, preferred_element_type=jnp.float32)
    idx_ref[0] = s0 + lax.broadcasted_iota(jnp.int32, (1, W), 1)

    @pl.when(jnp.max(ids) - s0 >= W)
    def _residual():
        rows = lax.broadcasted_iota(jnp.int32, (G, TILE), 0)
        ohr = ((rows == ids) & (local >= W)).astype(jnp.bfloat16)
        racc_ref[...] += jax.lax.dot(ohr, hb, preferred_element_type=jnp.float32)

    @pl.when(i == NB - 1)
    def _flush():
        resid_ref[...] = racc_ref[...]


def _segsum_body(part_hbm, idx_hbm, out_hbm, rows_v, idx_v, stage_v, acc_sh):
    cid = lax.axis_index("c")
    sid = lax.axis_index("s")
    wid = sid * NC + cid

    for r in range(ZROWS):
        for j in range(H // 16):
            stage_v[r, pl.ds(j * 16, 16)] = jnp.zeros((16,), jnp.float32)
    pltpu.sync_copy(stage_v, acc_sh.at[pl.ds(sid * ZROWS, ZROWS)])
    plsc.subcore_barrier()

    for k in range(KMAX):
        c = wid + NWORK * k

        @pl.when(c < NCHUNK)
        def _chunk():
            base = c * CH
            pltpu.sync_copy(idx_hbm.at[pl.ds(base, CH)], idx_v.at[k])
            pltpu.sync_copy(part_hbm.at[pl.ds(base, CH), :], rows_v)
            pltpu.sync_copy(rows_v, acc_sh.at[idx_v.at[k]], add=True)

    plsc.subcore_barrier()
    pltpu.sync_copy(acc_sh.at[pl.ds(sid * GROWS, GROWS)],
                    out_hbm.at[cid, pl.ds(sid * GROWS, GROWS)])


def _post_body(pp_ref, resid_ref, W3_ref, b3_ref, W4_ref, b4_ref, out_ref):
    p = pp_ref[0] + pp_ref[1] + resid_ref[...]
    g = jax.lax.dot(p, W3_ref[...], preferred_element_type=jnp.float32) + b3_ref[...]
    g = g * jax.nn.sigmoid(g)
    out_ref[...] = (jax.lax.dot(g, W4_ref[...], preferred_element_type=jnp.float32)
                    + b4_ref[...])


@jax.jit
def kernel(x, V, batch, W1, b1, W2, b2, W3, b3, W4, b4):
    v2 = V.reshape(N, 3 * NWIDTH).astype(jnp.bfloat16)
    batch_p = jnp.concatenate(
        [batch, jnp.full((NPADDED - N,), G, jnp.int32)]).reshape(NB, 1, TILE)

    full = lambda *s: pl.BlockSpec(s, lambda i: (0,) * len(s))
    parts, idxs, resid = pl.pallas_call(
        _pre_body,
        grid=(NB,),
        in_specs=[
            pl.BlockSpec((TILE, D), lambda i: (i, 0)),
            pl.BlockSpec((TILE, 3 * NWIDTH), lambda i: (i, 0)),
            pl.BlockSpec((1, 1, TILE), lambda i: (i, 0, 0)),
            full(D + NWIDTH, H),
            full(1, H),
            full(H, H),
            full(1, H),
        ],
        out_specs=[
            pl.BlockSpec((1, W, H), lambda i: (i, 0, 0)),
            pl.BlockSpec((1, 1, W), lambda i: (i, 0, 0)),
            pl.BlockSpec((G, H), lambda i: (0, 0)),
        ],
        out_shape=[
            jax.ShapeDtypeStruct((NB, W, H), jnp.float32),
            jax.ShapeDtypeStruct((NB, 1, W), jnp.int32),
            jax.ShapeDtypeStruct((G, H), jnp.float32),
        ],
        scratch_shapes=[pltpu.VMEM((G, H), jnp.float32)],
        compiler_params=pltpu.CompilerParams(
            dimension_semantics=("arbitrary",),
        ),
    )(x, v2, batch_p, W1, b1.reshape(1, H), W2, b2.reshape(1, H))

    mesh = plsc.VectorSubcoreMesh(core_axis_name="c", subcore_axis_name="s")
    segsum = functools.partial(
        pl.kernel,
        mesh=mesh,
        out_type=jax.ShapeDtypeStruct((NC, G, H), jnp.float32),
        scratch_types=[
            pltpu.VMEM((CH, H), jnp.float32),
            pltpu.VMEM((KMAX, CH), jnp.int32),
            pltpu.VMEM((ZROWS, H), jnp.float32),
            pltpu.VMEM_SHARED((ACCROWS, H), jnp.float32),
        ],
    )(_segsum_body)
    pooled2 = segsum(parts.reshape(NPART, H), idxs.reshape(NPART))

    out = pl.pallas_call(
        _post_body,
        in_specs=[
            pl.BlockSpec((NC, G, H), lambda: (0, 0, 0)),
            pl.BlockSpec((G, H), lambda: (0, 0)),
            pl.BlockSpec((H, H), lambda: (0, 0)),
            pl.BlockSpec((1, H), lambda: (0, 0)),
            pl.BlockSpec((H, 1), lambda: (0, 0)),
            pl.BlockSpec((1, 1), lambda: (0, 0)),
        ],
        out_specs=pl.BlockSpec((G, 1), lambda: (0, 0)),
        out_shape=jax.ShapeDtypeStruct((G, 1), jnp.float32),
    )(pooled2, resid, W3, b3.reshape(1, H), W4, b4.reshape(1, 1))
    return out


# speedup vs baseline: 3.2618x; 1.2225x over previous
import functools

import jax
import jax.numpy as jnp
from jax import lax
from jax.experimental import pallas as pl
from jax.experimental.pallas import tpu as pltpu
from jax.experimental.pallas import tpu_sc as plsc

N = 100000
D = 128
NWIDTH = 16
H = 128
G = 512

TILE = 1024
NB = -(-N // TILE)
NPADDED = NB * TILE
W = 64

CH = 128
NPART = NB * W
NCHUNK = NPART // CH
NC = 2
NS = 16
NWORK = NC * NS
KMAX = -(-NCHUNK // NWORK)
ACCROWS = 1024
ZROWS = ACCROWS // NS
GROWS = G // NS


def _pre_body(x_ref, v_ref, b_ref, W1_ref, b1_ref, W2_ref, b2_ref,
              part_ref, idx_ref, resid_ref, racc_ref):
    i = pl.program_id(0)

    @pl.when(i == 0)
    def _init():
        racc_ref[...] = jnp.zeros_like(racc_ref)

    xv = x_ref[...]
    vt = v_ref[...].astype(jnp.float32)
    invt = jnp.sqrt(vt[0:16, :] ** 2 + vt[16:32, :] ** 2 + vt[32:48, :] ** 2)
    h = (jax.lax.dot(xv.astype(jnp.bfloat16), W1_ref[0:D, :].astype(jnp.bfloat16),
                     preferred_element_type=jnp.float32)
         + jax.lax.dot_general(
             invt.astype(jnp.bfloat16), W1_ref[D:D + NWIDTH, :].astype(jnp.bfloat16),
             (((0,), (0,)), ((), ())), preferred_element_type=jnp.float32)
         + b1_ref[...])
    h = h * jax.nn.sigmoid(h)
    h = jax.lax.dot(h.astype(jnp.bfloat16), W2_ref[...].astype(jnp.bfloat16),
                    preferred_element_type=jnp.float32) + b2_ref[...]
    rvalid = lax.broadcasted_iota(jnp.int32, (TILE, 1), 0) + i * TILE
    h = jnp.where(rvalid < N, h, 0.0)
    hb = h.astype(jnp.bfloat16)

    ids = b_ref[0]
    s0 = ids[0, 0]
    local = ids - s0
    jwin = lax.broadcasted_iota(jnp.int32, (W, TILE), 0)
    ohw = (jwin == local).astype(jnp.bfloat16)
    part_ref[0] = jax.lax.dot(ohw, hb, preferred_element_type=jnp.float32)
    idx_ref[0] = s0 + lax.broadcasted_iota(jnp.int32, (1, W), 1)

    @pl.when(jnp.max(ids) - s0 >= W)
    def _residual():
        rows = lax.broadcasted_iota(jnp.int32, (G, TILE), 0)
        ohr = ((rows == ids) & (local >= W)).astype(jnp.bfloat16)
        racc_ref[...] += jax.lax.dot(ohr, hb, preferred_element_type=jnp.float32)

    @pl.when(i == NB - 1)
    def _flush():
        resid_ref[...] = racc_ref[...]


def _segsum_body(part_hbm, idx_hbm, out_hbm, rows_v, idx_v, stage_v, acc_sh):
    cid = lax.axis_index("c")
    sid = lax.axis_index("s")
    wid = sid * NC + cid

    for r in range(ZROWS):
        for j in range(H // 16):
            stage_v[r, pl.ds(j * 16, 16)] = jnp.zeros((16,), jnp.float32)
    pltpu.sync_copy(stage_v, acc_sh.at[pl.ds(sid * ZROWS, ZROWS)])
    plsc.subcore_barrier()

    for k in range(KMAX):
        c = wid + NWORK * k

        @pl.when(c < NCHUNK)
        def _chunk():
            base = c * CH
            pltpu.sync_copy(idx_hbm.at[pl.ds(base, CH)], idx_v.at[k])
            pltpu.sync_copy(part_hbm.at[pl.ds(base, CH), :], rows_v)
            pltpu.sync_copy(rows_v, acc_sh.at[idx_v.at[k]], add=True)

    plsc.subcore_barrier()
    pltpu.sync_copy(acc_sh.at[pl.ds(sid * GROWS, GROWS)],
                    out_hbm.at[cid, pl.ds(sid * GROWS, GROWS)])


def _post_body(pp_ref, resid_ref, W3_ref, b3_ref, W4_ref, b4_ref, out_ref):
    p = pp_ref[0] + pp_ref[1] + resid_ref[...]
    g = jax.lax.dot(p, W3_ref[...], preferred_element_type=jnp.float32) + b3_ref[...]
    g = g * jax.nn.sigmoid(g)
    out_ref[...] = (jax.lax.dot(g, W4_ref[...], preferred_element_type=jnp.float32)
                    + b4_ref[...])


@jax.jit
def kernel(x, V, batch, W1, b1, W2, b2, W3, b3, W4, b4):
    v2t = V.reshape(N, 3 * NWIDTH).astype(jnp.bfloat16).T
    batch_p = jnp.concatenate(
        [batch, jnp.full((NPADDED - N,), G, jnp.int32)]).reshape(NB, 1, TILE)

    full = lambda *s: pl.BlockSpec(s, lambda i: (0,) * len(s))
    parts, idxs, resid = pl.pallas_call(
        _pre_body,
        grid=(NB,),
        in_specs=[
            pl.BlockSpec((TILE, D), lambda i: (i, 0)),
            pl.BlockSpec((3 * NWIDTH, TILE), lambda i: (0, i)),
            pl.BlockSpec((1, 1, TILE), lambda i: (i, 0, 0)),
            full(D + NWIDTH, H),
            full(1, H),
            full(H, H),
            full(1, H),
        ],
        out_specs=[
            pl.BlockSpec((1, W, H), lambda i: (i, 0, 0)),
            pl.BlockSpec((1, 1, W), lambda i: (i, 0, 0)),
            pl.BlockSpec((G, H), lambda i: (0, 0)),
        ],
        out_shape=[
            jax.ShapeDtypeStruct((NB, W, H), jnp.float32),
            jax.ShapeDtypeStruct((NB, 1, W), jnp.int32),
            jax.ShapeDtypeStruct((G, H), jnp.float32),
        ],
        scratch_shapes=[pltpu.VMEM((G, H), jnp.float32)],
        compiler_params=pltpu.CompilerParams(
            dimension_semantics=("arbitrary",),
        ),
    )(x, v2t, batch_p, W1, b1.reshape(1, H), W2, b2.reshape(1, H))

    mesh = plsc.VectorSubcoreMesh(core_axis_name="c", subcore_axis_name="s")
    segsum = functools.partial(
        pl.kernel,
        mesh=mesh,
        out_type=jax.ShapeDtypeStruct((NC, G, H), jnp.float32),
        scratch_types=[
            pltpu.VMEM((CH, H), jnp.float32),
            pltpu.VMEM((KMAX, CH), jnp.int32),
            pltpu.VMEM((ZROWS, H), jnp.float32),
            pltpu.VMEM_SHARED((ACCROWS, H), jnp.float32),
        ],
    )(_segsum_body)
    pooled2 = segsum(parts.reshape(NPART, H), idxs.reshape(NPART))

    out = pl.pallas_call(
        _post_body,
        in_specs=[
            pl.BlockSpec((NC, G, H), lambda: (0, 0, 0)),
            pl.BlockSpec((G, H), lambda: (0, 0)),
            pl.BlockSpec((H, H), lambda: (0, 0)),
            pl.BlockSpec((1, H), lambda: (0, 0)),
            pl.BlockSpec((H, 1), lambda: (0, 0)),
            pl.BlockSpec((1, 1), lambda: (0, 0)),
        ],
        out_specs=pl.BlockSpec((G, 1), lambda: (0, 0)),
        out_shape=jax.ShapeDtypeStruct((G, 1), jnp.float32),
    )(pooled2, resid, W3, b3.reshape(1, H), W4, b4.reshape(1, 1))
    return out


# speedup vs baseline: 4.1315x; 1.2666x over previous
import functools

import jax
import jax.numpy as jnp
from jax import lax
from jax.experimental import pallas as pl
from jax.experimental.pallas import tpu as pltpu
from jax.experimental.pallas import tpu_sc as plsc

N = 100000
D = 128
NWIDTH = 16
H = 128
G = 512

TILE = 2048
NB = -(-N // TILE)
NPADDED = NB * TILE
W = 128

CH = 128
NPART = NB * W
NCHUNK = NPART // CH
NC = 2
NS = 16
NWORK = NC * NS
KMAX = -(-NCHUNK // NWORK)
ACCROWS = 1024
ZROWS = ACCROWS // NS
GROWS = G // NS


def _pre_body(x_ref, v_ref, b_ref, W1_ref, b1_ref, W2_ref, b2_ref,
              part_ref, idx_ref, resid_ref, racc_ref):
    i = pl.program_id(0)

    @pl.when(i == 0)
    def _init():
        racc_ref[...] = jnp.zeros_like(racc_ref)

    xv = x_ref[...]
    vt = v_ref[...].astype(jnp.float32)
    invt = jnp.sqrt(vt[0:16, :] ** 2 + vt[16:32, :] ** 2 + vt[32:48, :] ** 2)
    h = (jax.lax.dot(xv.astype(jnp.bfloat16), W1_ref[0:D, :].astype(jnp.bfloat16),
                     preferred_element_type=jnp.float32)
         + jax.lax.dot_general(
             invt.astype(jnp.bfloat16), W1_ref[D:D + NWIDTH, :].astype(jnp.bfloat16),
             (((0,), (0,)), ((), ())), preferred_element_type=jnp.float32)
         + b1_ref[...])
    h = h * jax.nn.sigmoid(h)
    h = jax.lax.dot(h.astype(jnp.bfloat16), W2_ref[...].astype(jnp.bfloat16),
                    preferred_element_type=jnp.float32) + b2_ref[...]
    rvalid = lax.broadcasted_iota(jnp.int32, (TILE, 1), 0) + i * TILE
    hb = jnp.where(rvalid < N, h.astype(jnp.bfloat16), jnp.bfloat16(0.0))

    ids = b_ref[0]
    s0 = ids[0, 0]
    local = ids - s0
    jwin = lax.broadcasted_iota(jnp.int32, (W, TILE), 0)
    ohw = (jwin == local).astype(jnp.bfloat16)
    part_ref[0] = jax.lax.dot(ohw, hb, preferred_element_type=jnp.float32)
    idx_ref[0] = s0 + lax.broadcasted_iota(jnp.int32, (1, W), 1)

    @pl.when(jnp.max(ids) - s0 >= W)
    def _residual():
        rows = lax.broadcasted_iota(jnp.int32, (G, TILE), 0)
        ohr = ((rows == ids) & (local >= W)).astype(jnp.bfloat16)
        racc_ref[...] += jax.lax.dot(ohr, hb, preferred_element_type=jnp.float32)

    @pl.when(i == NB - 1)
    def _flush():
        resid_ref[...] = racc_ref[...]


def _segsum_body(part_hbm, idx_hbm, out_hbm, rows_v, idx_v, stage_v, acc_sh):
    cid = lax.axis_index("c")
    sid = lax.axis_index("s")
    wid = sid * NC + cid

    for r in range(ZROWS):
        for j in range(H // 16):
            stage_v[r, pl.ds(j * 16, 16)] = jnp.zeros((16,), jnp.float32)
    pltpu.sync_copy(stage_v, acc_sh.at[pl.ds(sid * ZROWS, ZROWS)])
    plsc.subcore_barrier()

    for k in range(KMAX):
        c = wid + NWORK * k

        @pl.when(c < NCHUNK)
        def _chunk():
            base = c * CH
            pltpu.sync_copy(idx_hbm.at[pl.ds(base, CH)], idx_v.at[k])
            pltpu.sync_copy(part_hbm.at[pl.ds(base, CH), :], rows_v)
            pltpu.sync_copy(rows_v, acc_sh.at[idx_v.at[k]], add=True)

    plsc.subcore_barrier()
    pltpu.sync_copy(acc_sh.at[pl.ds(sid * GROWS, GROWS)],
                    out_hbm.at[cid, pl.ds(sid * GROWS, GROWS)])


def _post_body(pp_ref, resid_ref, W3_ref, b3_ref, W4_ref, b4_ref, out_ref):
    p = pp_ref[0] + pp_ref[1] + resid_ref[...]
    g = jax.lax.dot(p, W3_ref[...], preferred_element_type=jnp.float32) + b3_ref[...]
    g = g * jax.nn.sigmoid(g)
    out_ref[...] = (jax.lax.dot(g, W4_ref[...], preferred_element_type=jnp.float32)
                    + b4_ref[...])


@jax.jit
def kernel(x, V, batch, W1, b1, W2, b2, W3, b3, W4, b4):
    v2t = V.reshape(N, 3 * NWIDTH).astype(jnp.bfloat16).T
    batch_p = jnp.concatenate(
        [batch, jnp.full((NPADDED - N,), G, jnp.int32)]).reshape(NB, 1, TILE)

    full = lambda *s: pl.BlockSpec(s, lambda i: (0,) * len(s))
    parts, idxs, resid = pl.pallas_call(
        _pre_body,
        grid=(NB,),
        in_specs=[
            pl.BlockSpec((TILE, D), lambda i: (i, 0)),
            pl.BlockSpec((3 * NWIDTH, TILE), lambda i: (0, i)),
            pl.BlockSpec((1, 1, TILE), lambda i: (i, 0, 0)),
            full(D + NWIDTH, H),
            full(1, H),
            full(H, H),
            full(1, H),
        ],
        out_specs=[
            pl.BlockSpec((1, W, H), lambda i: (i, 0, 0)),
            pl.BlockSpec((1, 1, W), lambda i: (i, 0, 0)),
            pl.BlockSpec((G, H), lambda i: (0, 0)),
        ],
        out_shape=[
            jax.ShapeDtypeStruct((NB, W, H), jnp.float32),
            jax.ShapeDtypeStruct((NB, 1, W), jnp.int32),
            jax.ShapeDtypeStruct((G, H), jnp.float32),
        ],
        scratch_shapes=[pltpu.VMEM((G, H), jnp.float32)],
        compiler_params=pltpu.CompilerParams(
            dimension_semantics=("arbitrary",),
        ),
    )(x, v2t, batch_p, W1, b1.reshape(1, H), W2, b2.reshape(1, H))

    mesh = plsc.VectorSubcoreMesh(core_axis_name="c", subcore_axis_name="s")
    segsum = functools.partial(
        pl.kernel,
        mesh=mesh,
        out_type=jax.ShapeDtypeStruct((NC, G, H), jnp.float32),
        scratch_types=[
            pltpu.VMEM((CH, H), jnp.float32),
            pltpu.VMEM((KMAX, CH), jnp.int32),
            pltpu.VMEM((ZROWS, H), jnp.float32),
            pltpu.VMEM_SHARED((ACCROWS, H), jnp.float32),
        ],
    )(_segsum_body)
    pooled2 = segsum(parts.reshape(NPART, H), idxs.reshape(NPART))

    out = pl.pallas_call(
        _post_body,
        in_specs=[
            pl.BlockSpec((NC, G, H), lambda: (0, 0, 0)),
            pl.BlockSpec((G, H), lambda: (0, 0)),
            pl.BlockSpec((H, H), lambda: (0, 0)),
            pl.BlockSpec((1, H), lambda: (0, 0)),
            pl.BlockSpec((H, 1), lambda: (0, 0)),
            pl.BlockSpec((1, 1), lambda: (0, 0)),
        ],
        out_specs=pl.BlockSpec((G, 1), lambda: (0, 0)),
        out_shape=jax.ShapeDtypeStruct((G, 1), jnp.float32),
    )(pooled2, resid, W3, b3.reshape(1, H), W4, b4.reshape(1, 1))
    return out


# speedup vs baseline: 4.6512x; 1.1258x over previous
import functools

import jax
import jax.numpy as jnp
from jax import lax
from jax.experimental import pallas as pl
from jax.experimental.pallas import tpu as pltpu
from jax.experimental.pallas import tpu_sc as plsc

N = 100000
D = 128
NWIDTH = 16
H = 128
G = 512

TILE = 4096
NB = -(-N // TILE)
NPADDED = NB * TILE
W = 128

CH = 128
NPART = NB * W
NCHUNK = NPART // CH
NC = 2
NS = 16
NWORK = NC * NS
KMAX = -(-NCHUNK // NWORK)
ACCROWS = 1024
ZROWS = ACCROWS // NS
GROWS = G // NS


def _pre_body(x_ref, v_ref, b_ref, W1_ref, b1_ref, W2_ref, b2_ref,
              part_ref, idx_ref, resid_ref, racc_ref):
    i = pl.program_id(0)

    @pl.when(i == 0)
    def _init():
        racc_ref[...] = jnp.zeros_like(racc_ref)

    xv = x_ref[...]
    vt = v_ref[...].astype(jnp.float32)
    invt = jnp.sqrt(vt[0:16, :] ** 2 + vt[16:32, :] ** 2 + vt[32:48, :] ** 2)
    h = (jax.lax.dot(xv.astype(jnp.bfloat16), W1_ref[0:D, :].astype(jnp.bfloat16),
                     preferred_element_type=jnp.float32)
         + jax.lax.dot_general(
             invt.astype(jnp.bfloat16), W1_ref[D:D + NWIDTH, :].astype(jnp.bfloat16),
             (((0,), (0,)), ((), ())), preferred_element_type=jnp.float32)
         + b1_ref[...])
    h = h * jax.nn.sigmoid(h)
    h = jax.lax.dot(h.astype(jnp.bfloat16), W2_ref[...].astype(jnp.bfloat16),
                    preferred_element_type=jnp.float32) + b2_ref[...]
    rvalid = lax.broadcasted_iota(jnp.int32, (TILE, 1), 0) + i * TILE
    hb = jnp.where(rvalid < N, h.astype(jnp.bfloat16), jnp.bfloat16(0.0))

    ids = b_ref[0]
    s0 = ids[0, 0]
    local = ids - s0
    jwin = lax.broadcasted_iota(jnp.int32, (W, TILE), 0)
    ohw = (jwin == local).astype(jnp.bfloat16)
    part_ref[0] = jax.lax.dot(ohw, hb, preferred_element_type=jnp.float32)
    idx_ref[0] = s0 + lax.broadcasted_iota(jnp.int32, (1, W), 1)

    @pl.when(jnp.max(ids) - s0 >= W)
    def _residual():
        rows = lax.broadcasted_iota(jnp.int32, (G, TILE), 0)
        ohr = ((rows == ids) & (local >= W)).astype(jnp.bfloat16)
        racc_ref[...] += jax.lax.dot(ohr, hb, preferred_element_type=jnp.float32)

    @pl.when(i == NB - 1)
    def _flush():
        resid_ref[...] = racc_ref[...]


def _segsum_body(part_hbm, idx_hbm, out_hbm, rows_v, idx_v, stage_v, acc_sh):
    cid = lax.axis_index("c")
    sid = lax.axis_index("s")
    wid = sid * NC + cid

    for r in range(ZROWS):
        for j in range(H // 16):
            stage_v[r, pl.ds(j * 16, 16)] = jnp.zeros((16,), jnp.float32)
    pltpu.sync_copy(stage_v, acc_sh.at[pl.ds(sid * ZROWS, ZROWS)])
    plsc.subcore_barrier()

    for k in range(KMAX):
        c = wid + NWORK * k

        @pl.when(c < NCHUNK)
        def _chunk():
            base = c * CH
            pltpu.sync_copy(idx_hbm.at[pl.ds(base, CH)], idx_v.at[k])
            pltpu.sync_copy(part_hbm.at[pl.ds(base, CH), :], rows_v)
            pltpu.sync_copy(rows_v, acc_sh.at[idx_v.at[k]], add=True)

    plsc.subcore_barrier()
    pltpu.sync_copy(acc_sh.at[pl.ds(sid * GROWS, GROWS)],
                    out_hbm.at[cid, pl.ds(sid * GROWS, GROWS)])


def _post_body(pp_ref, resid_ref, W3_ref, b3_ref, W4_ref, b4_ref, out_ref):
    p = pp_ref[0] + pp_ref[1] + resid_ref[...]
    g = jax.lax.dot(p, W3_ref[...], preferred_element_type=jnp.float32) + b3_ref[...]
    g = g * jax.nn.sigmoid(g)
    out_ref[...] = (jax.lax.dot(g, W4_ref[...], preferred_element_type=jnp.float32)
                    + b4_ref[...])


@jax.jit
def kernel(x, V, batch, W1, b1, W2, b2, W3, b3, W4, b4):
    v2t = V.reshape(N, 3 * NWIDTH).astype(jnp.bfloat16).T
    batch_p = jnp.concatenate(
        [batch, jnp.full((NPADDED - N,), G, jnp.int32)]).reshape(NB, 1, TILE)

    full = lambda *s: pl.BlockSpec(s, lambda i: (0,) * len(s))
    parts, idxs, resid = pl.pallas_call(
        _pre_body,
        grid=(NB,),
        in_specs=[
            pl.BlockSpec((TILE, D), lambda i: (i, 0)),
            pl.BlockSpec((3 * NWIDTH, TILE), lambda i: (0, i)),
            pl.BlockSpec((1, 1, TILE), lambda i: (i, 0, 0)),
            full(D + NWIDTH, H),
            full(1, H),
            full(H, H),
            full(1, H),
        ],
        out_specs=[
            pl.BlockSpec((1, W, H), lambda i: (i, 0, 0)),
            pl.BlockSpec((1, 1, W), lambda i: (i, 0, 0)),
            pl.BlockSpec((G, H), lambda i: (0, 0)),
        ],
        out_shape=[
            jax.ShapeDtypeStruct((NB, W, H), jnp.float32),
            jax.ShapeDtypeStruct((NB, 1, W), jnp.int32),
            jax.ShapeDtypeStruct((G, H), jnp.float32),
        ],
        scratch_shapes=[pltpu.VMEM((G, H), jnp.float32)],
        compiler_params=pltpu.CompilerParams(
            dimension_semantics=("arbitrary",),
        ),
    )(x, v2t, batch_p, W1, b1.reshape(1, H), W2, b2.reshape(1, H))

    mesh = plsc.VectorSubcoreMesh(core_axis_name="c", subcore_axis_name="s")
    segsum = functools.partial(
        pl.kernel,
        mesh=mesh,
        out_type=jax.ShapeDtypeStruct((NC, G, H), jnp.float32),
        scratch_types=[
            pltpu.VMEM((CH, H), jnp.float32),
            pltpu.VMEM((KMAX, CH), jnp.int32),
            pltpu.VMEM((ZROWS, H), jnp.float32),
            pltpu.VMEM_SHARED((ACCROWS, H), jnp.float32),
        ],
    )(_segsum_body)
    pooled2 = segsum(parts.reshape(NPART, H), idxs.reshape(NPART))

    out = pl.pallas_call(
        _post_body,
        in_specs=[
            pl.BlockSpec((NC, G, H), lambda: (0, 0, 0)),
            pl.BlockSpec((G, H), lambda: (0, 0)),
            pl.BlockSpec((H, H), lambda: (0, 0)),
            pl.BlockSpec((1, H), lambda: (0, 0)),
            pl.BlockSpec((H, 1), lambda: (0, 0)),
            pl.BlockSpec((1, 1), lambda: (0, 0)),
        ],
        out_specs=pl.BlockSpec((G, 1), lambda: (0, 0)),
        out_shape=jax.ShapeDtypeStruct((G, 1), jnp.float32),
    )(pooled2, resid, W3, b3.reshape(1, H), W4, b4.reshape(1, 1))
    return out


# speedup vs baseline: 4.7427x; 1.0197x over previous
import functools

import jax
import jax.numpy as jnp
from jax import lax
from jax.experimental import pallas as pl
from jax.experimental.pallas import tpu as pltpu
from jax.experimental.pallas import tpu_sc as plsc

N = 100000
D = 128
NWIDTH = 16
H = 128
G = 512

TILE = 4096
NB = -(-N // TILE)
NPADDED = NB * TILE
W = 128

CH = 128
NPART = NB * W
NCHUNK = NPART // CH
NC = 2
NS = 16
NWORK = NC * NS
KMAX = -(-NCHUNK // NWORK)
ACCROWS = 1024
ZROWS = ACCROWS // NS
GROWS = G // NS


def _pre_body(x_ref, v_ref, b_ref, W1_ref, b1_ref, W2_ref, b2_ref,
              part_ref, idx_ref, resid_ref, racc_ref):
    i = pl.program_id(0)

    @pl.when(i == 0)
    def _init():
        racc_ref[...] = jnp.zeros_like(racc_ref)

    xv = x_ref[...]
    vt = v_ref[...].astype(jnp.float32)
    invt = jnp.sqrt(vt[0:16, :] ** 2 + vt[16:32, :] ** 2 + vt[32:48, :] ** 2)
    h = (jax.lax.dot(xv, W1_ref[0:D, :], preferred_element_type=jnp.float32)
         + jax.lax.dot_general(
             invt, W1_ref[D:D + NWIDTH, :],
             (((0,), (0,)), ((), ())), preferred_element_type=jnp.float32)
         + b1_ref[...])
    h = h * jax.nn.sigmoid(h)
    h = jax.lax.dot(h, W2_ref[...], preferred_element_type=jnp.float32) + b2_ref[...]
    rvalid = lax.broadcasted_iota(jnp.int32, (TILE, 1), 0) + i * TILE
    hb = jnp.where(rvalid < N, h, 0.0)

    ids = b_ref[0]
    s0 = ids[0, 0]
    local = ids - s0
    jwin = lax.broadcasted_iota(jnp.int32, (W, TILE), 0)
    ohw = (jwin == local).astype(jnp.float32)
    part_ref[0] = jax.lax.dot(ohw, hb, preferred_element_type=jnp.float32)
    idx_ref[0] = s0 + lax.broadcasted_iota(jnp.int32, (1, W), 1)

    @pl.when(jnp.max(ids) - s0 >= W)
    def _residual():
        rows = lax.broadcasted_iota(jnp.int32, (G, TILE), 0)
        ohr = ((rows == ids) & (local >= W)).astype(jnp.float32)
        racc_ref[...] += jax.lax.dot(ohr, hb, preferred_element_type=jnp.float32)

    @pl.when(i == NB - 1)
    def _flush():
        resid_ref[...] = racc_ref[...]


def _segsum_body(part_hbm, idx_hbm, out_hbm, rows_v, idx_v, stage_v, acc_sh):
    cid = lax.axis_index("c")
    sid = lax.axis_index("s")
    wid = sid * NC + cid

    for r in range(ZROWS):
        for j in range(H // 16):
            stage_v[r, pl.ds(j * 16, 16)] = jnp.zeros((16,), jnp.float32)
    pltpu.sync_copy(stage_v, acc_sh.at[pl.ds(sid * ZROWS, ZROWS)])
    plsc.subcore_barrier()

    for k in range(KMAX):
        c = wid + NWORK * k

        @pl.when(c < NCHUNK)
        def _chunk():
            base = c * CH
            pltpu.sync_copy(idx_hbm.at[pl.ds(base, CH)], idx_v.at[k])
            pltpu.sync_copy(part_hbm.at[pl.ds(base, CH), :], rows_v)
            pltpu.sync_copy(rows_v, acc_sh.at[idx_v.at[k]], add=True)

    plsc.subcore_barrier()
    pltpu.sync_copy(acc_sh.at[pl.ds(sid * GROWS, GROWS)],
                    out_hbm.at[cid, pl.ds(sid * GROWS, GROWS)])


def _post_body(pp_ref, resid_ref, W3_ref, b3_ref, W4_ref, b4_ref, out_ref):
    p = pp_ref[0] + pp_ref[1] + resid_ref[...]
    g = jax.lax.dot(p, W3_ref[...], preferred_element_type=jnp.float32) + b3_ref[...]
    g = g * jax.nn.sigmoid(g)
    out_ref[...] = (jax.lax.dot(g, W4_ref[...], preferred_element_type=jnp.float32)
                    + b4_ref[...])


@jax.jit
def kernel(x, V, batch, W1, b1, W2, b2, W3, b3, W4, b4):
    v2t = V.reshape(N, 3 * NWIDTH).astype(jnp.bfloat16).T
    batch_p = jnp.concatenate(
        [batch, jnp.full((NPADDED - N,), G, jnp.int32)]).reshape(NB, 1, TILE)

    full = lambda *s: pl.BlockSpec(s, lambda i: (0,) * len(s))
    parts, idxs, resid = pl.pallas_call(
        _pre_body,
        grid=(NB,),
        in_specs=[
            pl.BlockSpec((TILE, D), lambda i: (i, 0)),
            pl.BlockSpec((3 * NWIDTH, TILE), lambda i: (0, i)),
            pl.BlockSpec((1, 1, TILE), lambda i: (i, 0, 0)),
            full(D + NWIDTH, H),
            full(1, H),
            full(H, H),
            full(1, H),
        ],
        out_specs=[
            pl.BlockSpec((1, W, H), lambda i: (i, 0, 0)),
            pl.BlockSpec((1, 1, W), lambda i: (i, 0, 0)),
            pl.BlockSpec((G, H), lambda i: (0, 0)),
        ],
        out_shape=[
            jax.ShapeDtypeStruct((NB, W, H), jnp.float32),
            jax.ShapeDtypeStruct((NB, 1, W), jnp.int32),
            jax.ShapeDtypeStruct((G, H), jnp.float32),
        ],
        scratch_shapes=[pltpu.VMEM((G, H), jnp.float32)],
        compiler_params=pltpu.CompilerParams(
            dimension_semantics=("arbitrary",),
        ),
    )(x, v2t, batch_p, W1, b1.reshape(1, H), W2, b2.reshape(1, H))

    mesh = plsc.VectorSubcoreMesh(core_axis_name="c", subcore_axis_name="s")
    segsum = functools.partial(
        pl.kernel,
        mesh=mesh,
        out_type=jax.ShapeDtypeStruct((NC, G, H), jnp.float32),
        scratch_types=[
            pltpu.VMEM((CH, H), jnp.float32),
            pltpu.VMEM((KMAX, CH), jnp.int32),
            pltpu.VMEM((ZROWS, H), jnp.float32),
            pltpu.VMEM_SHARED((ACCROWS, H), jnp.float32),
        ],
    )(_segsum_body)
    pooled2 = segsum(parts.reshape(NPART, H), idxs.reshape(NPART))

    out = pl.pallas_call(
        _post_body,
        in_specs=[
            pl.BlockSpec((NC, G, H), lambda: (0, 0, 0)),
            pl.BlockSpec((G, H), lambda: (0, 0)),
            pl.BlockSpec((H, H), lambda: (0, 0)),
            pl.BlockSpec((1, H), lambda: (0, 0)),
            pl.BlockSpec((H, 1), lambda: (0, 0)),
            pl.BlockSpec((1, 1), lambda: (0, 0)),
        ],
        out_specs=pl.BlockSpec((G, 1), lambda: (0, 0)),
        out_shape=jax.ShapeDtypeStruct((G, 1), jnp.float32),
    )(pooled2, resid, W3, b3.reshape(1, H), W4, b4.reshape(1, 1))
    return out


# speedup vs baseline: 5.3056x; 1.1187x over previous
import functools

import jax
import jax.numpy as jnp
from jax import lax
from jax.experimental import pallas as pl
from jax.experimental.pallas import tpu as pltpu
from jax.experimental.pallas import tpu_sc as plsc

N = 100000
D = 128
NWIDTH = 16
H = 128
G = 512

TILE = 4096
NB = -(-N // TILE)
NPADDED = NB * TILE
W = 128

CH = 128
NPART = NB * W
NCHUNK = NPART // CH
NC = 2
NS = 16
NWORK = NC * NS
KMAX = -(-NCHUNK // NWORK)
ACCROWS = 1024
ZROWS = ACCROWS // NS
GROWS = G // NS


def _pre_body(x_ref, v_ref, b_ref, W1_ref, b1_ref, W2_ref, b2_ref,
              part_ref, idx_ref, resid_ref, racc_ref):
    i = pl.program_id(0)

    @pl.when(i == 0)
    def _init():
        racc_ref[...] = jnp.zeros_like(racc_ref)

    xv = x_ref[...]
    vt = v_ref[...]
    invt = jnp.sqrt(vt[0:16, :] ** 2 + vt[16:32, :] ** 2 + vt[32:48, :] ** 2)
    h = (jax.lax.dot(xv, W1_ref[0:D, :], preferred_element_type=jnp.float32)
         + jax.lax.dot_general(
             invt, W1_ref[D:D + NWIDTH, :],
             (((0,), (0,)), ((), ())), preferred_element_type=jnp.float32)
         + b1_ref[...])
    h = h * jax.nn.sigmoid(h)
    h = jax.lax.dot(h, W2_ref[...], preferred_element_type=jnp.float32) + b2_ref[...]
    rvalid = lax.broadcasted_iota(jnp.int32, (TILE, 1), 0) + i * TILE
    hb = jnp.where(rvalid < N, h, 0.0)

    ids = b_ref[0]
    s0 = ids[0, 0]
    local = ids - s0
    jwin = lax.broadcasted_iota(jnp.int32, (W, TILE), 0)
    ohw = (jwin == local).astype(jnp.float32)
    part_ref[0] = jax.lax.dot(ohw, hb, preferred_element_type=jnp.float32)
    idx_ref[0] = s0 + lax.broadcasted_iota(jnp.int32, (1, W), 1)

    @pl.when(jnp.max(ids) - s0 >= W)
    def _residual():
        rows = lax.broadcasted_iota(jnp.int32, (G, TILE), 0)
        ohr = ((rows == ids) & (local >= W)).astype(jnp.float32)
        racc_ref[...] += jax.lax.dot(ohr, hb, preferred_element_type=jnp.float32)

    @pl.when(i == NB - 1)
    def _flush():
        resid_ref[...] = racc_ref[...]


def _segsum_body(part_hbm, idx_hbm, out_hbm, rows_v, idx_v, stage_v, acc_sh):
    cid = lax.axis_index("c")
    sid = lax.axis_index("s")
    wid = sid * NC + cid

    for r in range(ZROWS):
        for j in range(H // 16):
            stage_v[r, pl.ds(j * 16, 16)] = jnp.zeros((16,), jnp.float32)
    pltpu.sync_copy(stage_v, acc_sh.at[pl.ds(sid * ZROWS, ZROWS)])
    plsc.subcore_barrier()

    for k in range(KMAX):
        c = wid + NWORK * k

        @pl.when(c < NCHUNK)
        def _chunk():
            base = c * CH
            pltpu.sync_copy(idx_hbm.at[pl.ds(base, CH)], idx_v.at[k])
            pltpu.sync_copy(part_hbm.at[pl.ds(base, CH), :], rows_v)
            pltpu.sync_copy(rows_v, acc_sh.at[idx_v.at[k]], add=True)

    plsc.subcore_barrier()
    pltpu.sync_copy(acc_sh.at[pl.ds(sid * GROWS, GROWS)],
                    out_hbm.at[cid, pl.ds(sid * GROWS, GROWS)])


def _post_body(pp_ref, resid_ref, W3_ref, b3_ref, W4_ref, b4_ref, out_ref):
    p = pp_ref[0] + pp_ref[1] + resid_ref[...]
    g = jax.lax.dot(p, W3_ref[...], preferred_element_type=jnp.float32) + b3_ref[...]
    g = g * jax.nn.sigmoid(g)
    out_ref[...] = (jax.lax.dot(g, W4_ref[...], preferred_element_type=jnp.float32)
                    + b4_ref[...])


@jax.jit
def kernel(x, V, batch, W1, b1, W2, b2, W3, b3, W4, b4):
    v2t = V.reshape(N, 3 * NWIDTH).T
    batch_p = jnp.concatenate(
        [batch, jnp.full((NPADDED - N,), G, jnp.int32)]).reshape(NB, 1, TILE)

    full = lambda *s: pl.BlockSpec(s, lambda i: (0,) * len(s))
    parts, idxs, resid = pl.pallas_call(
        _pre_body,
        grid=(NB,),
        in_specs=[
            pl.BlockSpec((TILE, D), lambda i: (i, 0)),
            pl.BlockSpec((3 * NWIDTH, TILE), lambda i: (0, i)),
            pl.BlockSpec((1, 1, TILE), lambda i: (i, 0, 0)),
            full(D + NWIDTH, H),
            full(1, H),
            full(H, H),
            full(1, H),
        ],
        out_specs=[
            pl.BlockSpec((1, W, H), lambda i: (i, 0, 0)),
            pl.BlockSpec((1, 1, W), lambda i: (i, 0, 0)),
            pl.BlockSpec((G, H), lambda i: (0, 0)),
        ],
        out_shape=[
            jax.ShapeDtypeStruct((NB, W, H), jnp.float32),
            jax.ShapeDtypeStruct((NB, 1, W), jnp.int32),
            jax.ShapeDtypeStruct((G, H), jnp.float32),
        ],
        scratch_shapes=[pltpu.VMEM((G, H), jnp.float32)],
        compiler_params=pltpu.CompilerParams(
            dimension_semantics=("arbitrary",),
        ),
    )(x, v2t, batch_p, W1, b1.reshape(1, H), W2, b2.reshape(1, H))

    mesh = plsc.VectorSubcoreMesh(core_axis_name="c", subcore_axis_name="s")
    segsum = functools.partial(
        pl.kernel,
        mesh=mesh,
        out_type=jax.ShapeDtypeStruct((NC, G, H), jnp.float32),
        scratch_types=[
            pltpu.VMEM((CH, H), jnp.float32),
            pltpu.VMEM((KMAX, CH), jnp.int32),
            pltpu.VMEM((ZROWS, H), jnp.float32),
            pltpu.VMEM_SHARED((ACCROWS, H), jnp.float32),
        ],
    )(_segsum_body)
    pooled2 = segsum(parts.reshape(NPART, H), idxs.reshape(NPART))

    out = pl.pallas_call(
        _post_body,
        in_specs=[
            pl.BlockSpec((NC, G, H), lambda: (0, 0, 0)),
            pl.BlockSpec((G, H), lambda: (0, 0)),
            pl.BlockSpec((H, H), lambda: (0, 0)),
            pl.BlockSpec((1, H), lambda: (0, 0)),
            pl.BlockSpec((H, 1), lambda: (0, 0)),
            pl.BlockSpec((1, 1), lambda: (0, 0)),
        ],
        out_specs=pl.BlockSpec((G, 1), lambda: (0, 0)),
        out_shape=jax.ShapeDtypeStruct((G, 1), jnp.float32),
    )(pooled2, resid, W3, b3.reshape(1, H), W4, b4.reshape(1, 1))
    return out
